# Initial kernel scaffold; baseline (speedup 1.0000x reference)
import jax
import jax.numpy as jnp
from jax.experimental import pallas as pl


def kernel(x, edge_index, edge_attr, params):
    raise NotImplementedError("write your pallas kernel here")


# speedup vs baseline: 1.0672x; 1.0672x over previous
import functools

import jax
import jax.numpy as jnp
from jax.experimental import pallas as pl
from jax.experimental.pallas import tpu as pltpu

N = 10000
E = 160000
H = 64
BN = 1000
BE = 2000

_INTERPRET = False


def _flatten_mlps(mlps):
    flat, counts = [], []
    for layers in mlps:
        counts.append(len(layers))
        for W, b in layers:
            flat.append(W)
            flat.append(b.reshape(1, -1))
    return flat, counts


def _read_mlps(refs, counts):
    out, k = [], 0
    for c in counts:
        layers = []
        for _ in range(c):
            layers.append((refs[k][...], refs[k + 1][...]))
            k += 2
        out.append(layers)
    return out


def _dot(x, W):
    return jnp.dot(x, W, preferred_element_type=jnp.float32)


def _apply(x, layers):
    n = len(layers)
    for i, (W, b) in enumerate(layers):
        x = _dot(x, W) + b
        if i < n - 1:
            x = jnp.maximum(x, 0.0)
    return x


def _row_spec(blk, cols):
    return pl.BlockSpec((blk, cols), lambda i: (i, 0))


def _w_spec(shape):
    nd = len(shape)
    return pl.BlockSpec(shape, lambda i: (0,) * nd)


def _rows_call(body, n_rows, blk, row_ins, weights, out_cols):
    grid = n_rows // blk
    in_specs = ([_row_spec(blk, a.shape[1]) for a in row_ins]
                + [_w_spec(w.shape) for w in weights])
    out_specs = [_row_spec(blk, c) for c in out_cols]
    out_shape = [jax.ShapeDtypeStruct((n_rows, c), jnp.float32) for c in out_cols]
    return pl.pallas_call(
        body,
        grid=(grid,),
        in_specs=in_specs,
        out_specs=out_specs,
        out_shape=out_shape,
        interpret=_INTERPRET,
    )(*row_ins, *weights)


def _enc_node_body(counts, x_ref, *refs):
    wrefs, outs = refs[:-3], refs[-3:]
    enc, snd, rcv = _read_mlps(wrefs, counts)
    node = _apply(x_ref[...], enc)
    outs[0][...] = node
    outs[1][...] = _apply(node, snd)
    outs[2][...] = _apply(node, rcv)


def _enc_edge_body(counts, ea_ref, *refs):
    wrefs, outs = refs[:-2], refs[-2:]
    enc, emlp = _read_mlps(wrefs, counts)
    edge = _apply(ea_ref[...], enc)
    outs[0][...] = edge
    outs[1][...] = _apply(edge, emlp)


def _edge_update_body(counts, eprev_ref, m_ref, *refs):
    wrefs, outs = refs[:-2], refs[-2:]
    (emlp,) = _read_mlps(wrefs, counts)
    edge = eprev_ref[...] + m_ref[...]
    outs[0][...] = edge
    outs[1][...] = _apply(edge, emlp)


def _node_update_body(counts, node_ref, agg_ref, *refs):
    wrefs, outs = refs[:-3], refs[-3:]
    nmlp, snd, rcv = _read_mlps(wrefs, counts)
    node = node_ref[...]
    cat = jnp.concatenate([node, agg_ref[...]], axis=1)
    nn = _apply(cat, nmlp)
    node = nn + node
    outs[0][...] = node
    outs[1][...] = _apply(node, snd)
    outs[2][...] = _apply(node, rcv)


def _node_dec_body(counts, node_ref, agg_ref, *refs):
    wrefs, outs = refs[:-1], refs[-1:]
    nmlp, dec = _read_mlps(wrefs, counts)
    node = node_ref[...]
    cat = jnp.concatenate([node, agg_ref[...]], axis=1)
    nn = _apply(cat, nmlp)
    node = nn + node
    outs[0][...] = _apply(node, dec)


def _message_pass(S, R, T, senders, receivers):
    m = jnp.take(S, senders, axis=0) + jnp.take(R, receivers, axis=0) + T
    agg = jax.ops.segment_sum(m, receivers, num_segments=N)
    return m, agg


def kernel(x, edge_index, edge_attr, params):
    senders = edge_index[0]
    receivers = edge_index[1]
    blocks = params["blocks"]

    flat, counts = _flatten_mlps(
        [params["enc_node"], blocks[0]["sender"], blocks[0]["receiver"]])
    node, S, R = _rows_call(functools.partial(_enc_node_body, counts),
                            N, BN, [x], flat, [H, H, H])

    flat, counts = _flatten_mlps([params["enc_edge"], blocks[0]["edge"]])
    edge, T = _rows_call(functools.partial(_enc_edge_body, counts),
                         E, BE, [edge_attr], flat, [H, H])

    out = None
    for i in range(5):
        m, agg = _message_pass(S, R, T, senders, receivers)
        if i < 4:
            flat, counts = _flatten_mlps(
                [blocks[i]["node"],
                 blocks[i + 1]["sender"], blocks[i + 1]["receiver"]])
            node, S, R = _rows_call(functools.partial(_node_update_body, counts),
                                    N, BN, [node, agg], flat, [H, H, H])
            flat, counts = _flatten_mlps([blocks[i + 1]["edge"]])
            edge, T = _rows_call(functools.partial(_edge_update_body, counts),
                                 E, BE, [edge, m], flat, [H, H])
        else:
            flat, counts = _flatten_mlps(
                [blocks[i]["node"], params["dec"]])
            (out,) = _rows_call(functools.partial(_node_dec_body, counts),
                                N, BN, [node, agg], flat, [2])
    return out


# speedup vs baseline: 1.0672x; 1.0001x over previous
import functools

import jax
import jax.numpy as jnp
from jax.experimental import pallas as pl
from jax.experimental.pallas import tpu as pltpu

N = 10000
E = 160000
H = 64
BN = 1000
BE = 2000


def _flatten_mlps(mlps):
    flat, counts = [], []
    for layers in mlps:
        counts.append(len(layers))
        for W, b in layers:
            flat.append(W)
            flat.append(b.reshape(1, -1))
    return flat, counts


def _read_mlps(refs, counts):
    out, k = [], 0
    for c in counts:
        layers = []
        for _ in range(c):
            layers.append((refs[k][...], refs[k + 1][...]))
            k += 2
        out.append(layers)
    return out


def _dot(x, W):
    return jnp.dot(x, W, preferred_element_type=jnp.float32)


def _apply(x, layers):
    n = len(layers)
    for i, (W, b) in enumerate(layers):
        x = _dot(x, W) + b
        if i < n - 1:
            x = jnp.maximum(x, 0.0)
    return x


def _row_spec(blk, cols):
    return pl.BlockSpec((blk, cols), lambda i: (i, 0))


def _w_spec(shape):
    nd = len(shape)
    return pl.BlockSpec(shape, lambda i: (0,) * nd)


def _rows_call(body, n_rows, blk, row_ins, weights, out_cols):
    grid = n_rows // blk
    in_specs = ([_row_spec(blk, a.shape[1]) for a in row_ins]
                + [_w_spec(w.shape) for w in weights])
    out_specs = [_row_spec(blk, c) for c in out_cols]
    out_shape = [jax.ShapeDtypeStruct((n_rows, c), jnp.float32) for c in out_cols]
    return pl.pallas_call(
        body,
        grid=(grid,),
        in_specs=in_specs,
        out_specs=out_specs,
        out_shape=out_shape,
    )(*row_ins, *weights)


def _enc_node_body(counts, x_ref, *refs):
    wrefs, outs = refs[:-3], refs[-3:]
    enc, snd, rcv = _read_mlps(wrefs, counts)
    node = _apply(x_ref[...], enc)
    outs[0][...] = node
    outs[1][...] = _apply(node, snd)
    outs[2][...] = _apply(node, rcv)


def _enc_edge_body(counts, ea_ref, *refs):
    wrefs, outs = refs[:-2], refs[-2:]
    enc, emlp = _read_mlps(wrefs, counts)
    edge = _apply(ea_ref[...], enc)
    outs[0][...] = edge
    outs[1][...] = _apply(edge, emlp)


def _edge_update_body(counts, eprev_ref, m_ref, *refs):
    wrefs, outs = refs[:-2], refs[-2:]
    (emlp,) = _read_mlps(wrefs, counts)
    edge = eprev_ref[...] + m_ref[...]
    outs[0][...] = edge
    outs[1][...] = _apply(edge, emlp)


def _node_update_body(counts, node_ref, agg_ref, *refs):
    wrefs, outs = refs[:-3], refs[-3:]
    nmlp, snd, rcv = _read_mlps(wrefs, counts)
    node = node_ref[...]
    cat = jnp.concatenate([node, agg_ref[...]], axis=1)
    nn = _apply(cat, nmlp)
    node = nn + node
    outs[0][...] = node
    outs[1][...] = _apply(node, snd)
    outs[2][...] = _apply(node, rcv)


def _node_dec_body(counts, node_ref, agg_ref, *refs):
    wrefs, outs = refs[:-1], refs[-1:]
    nmlp, dec = _read_mlps(wrefs, counts)
    node = node_ref[...]
    cat = jnp.concatenate([node, agg_ref[...]], axis=1)
    nn = _apply(cat, nmlp)
    node = nn + node
    outs[0][...] = _apply(node, dec)


def _message_pass(S, R, T, senders, receivers):
    m = jnp.take(S, senders, axis=0) + jnp.take(R, receivers, axis=0) + T
    agg = jax.ops.segment_sum(m, receivers, num_segments=N)
    return m, agg


def kernel(x, edge_index, edge_attr, params):
    senders = edge_index[0]
    receivers = edge_index[1]
    blocks = params["blocks"]

    flat, counts = _flatten_mlps(
        [params["enc_node"], blocks[0]["sender"], blocks[0]["receiver"]])
    node, S, R = _rows_call(functools.partial(_enc_node_body, counts),
                            N, BN, [x], flat, [H, H, H])

    flat, counts = _flatten_mlps([params["enc_edge"], blocks[0]["edge"]])
    edge, T = _rows_call(functools.partial(_enc_edge_body, counts),
                         E, BE, [edge_attr], flat, [H, H])

    out = None
    for i in range(5):
        m, agg = _message_pass(S, R, T, senders, receivers)
        if i < 4:
            flat, counts = _flatten_mlps(
                [blocks[i]["node"],
                 blocks[i + 1]["sender"], blocks[i + 1]["receiver"]])
            node, S, R = _rows_call(functools.partial(_node_update_body, counts),
                                    N, BN, [node, agg], flat, [H, H, H])
            flat, counts = _flatten_mlps([blocks[i + 1]["edge"]])
            edge, T = _rows_call(functools.partial(_edge_update_body, counts),
                                 E, BE, [edge, m], flat, [H, H])
        else:
            flat, counts = _flatten_mlps(
                [blocks[i]["node"], params["dec"]])
            (out,) = _rows_call(functools.partial(_node_dec_body, counts),
                                N, BN, [node, agg], flat, [2])
    return out
